# Initial kernel scaffold; baseline (speedup 1.0000x reference)
import functools

import jax
import jax.numpy as jnp
from jax import lax
from jax.experimental import pallas as pl
from jax.experimental.pallas import tpu as pltpu
from jax.experimental.pallas import tpu_sc as plsc

N = 10000
E = 320000
D = 128
D_OUT = 10
G = 64

NC = 2
NS = 16
LANES = 16
NW = NC * NS
CHUNK = 128
RPW = 79
E_PAD = NW * RPW * CHUNK
N_ACC = 10016
DRAIN = N_ACC // NS

NB = 1000
GRID = N // NB

_mesh = plsc.VectorSubcoreMesh(core_axis_name="c", subcore_axis_name="s")


@functools.partial(
    pl.kernel,
    out_type=jax.ShapeDtypeStruct((NC, N_ACC, D), jnp.float32),
    mesh=_mesh,
    scratch_types=[
        pltpu.VMEM((RPW, CHUNK), jnp.int32),
        pltpu.VMEM((RPW, CHUNK), jnp.int32),
        pltpu.VMEM((CHUNK, D), jnp.float32),
        pltpu.VMEM_SHARED((N_ACC, D), jnp.float32),
        pltpu.SemaphoreType.DMA,
    ],
)
def _sc_scatter(ht_hbm, srcp_hbm, dstp_hbm, zeros_hbm, out_hbm,
                srcv, dstv, buf, accum, sem):
    c = lax.axis_index("c")
    s = lax.axis_index("s")
    wid = s * NC + c
    base = wid * RPW
    pltpu.sync_copy(srcp_hbm.at[pl.ds(base, RPW)], srcv)
    pltpu.sync_copy(dstp_hbm.at[pl.ds(base, RPW)], dstv)
    row0 = s * DRAIN
    pltpu.sync_copy(zeros_hbm.at[pl.ds(row0, DRAIN)], accum.at[pl.ds(row0, DRAIN)])
    plsc.subcore_barrier()

    @pl.loop(0, RPW)
    def _(j):
        pltpu.async_copy(ht_hbm.at[srcv.at[j]], buf, sem).wait()
        pltpu.sync_copy(buf, accum.at[dstv.at[j]], add=True)

    plsc.subcore_barrier()
    pltpu.sync_copy(accum.at[pl.ds(row0, DRAIN)], out_hbm.at[c, pl.ds(row0, DRAIN)])


@functools.partial(
    pl.kernel,
    out_type=jax.ShapeDtypeStruct((NC, N_ACC, LANES), jnp.float32),
    mesh=_mesh,
    scratch_types=[
        pltpu.VMEM((RPW, CHUNK), jnp.int32),
        pltpu.VMEM((CHUNK, LANES), jnp.float32),
        pltpu.VMEM_SHARED((N_ACC, LANES), jnp.float32),
    ],
)
def _sc_degree(dstp_hbm, zeros_hbm, out_hbm, dstv, ones, accum):
    c = lax.axis_index("c")
    s = lax.axis_index("s")
    wid = s * NC + c
    pltpu.sync_copy(dstp_hbm.at[pl.ds(wid * RPW, RPW)], dstv)

    @pl.loop(0, CHUNK)
    def _(i):
        ones[i, :] = jnp.full((LANES,), 1.0, jnp.float32)

    row0 = s * DRAIN
    pltpu.sync_copy(zeros_hbm.at[pl.ds(row0, DRAIN)], accum.at[pl.ds(row0, DRAIN)])
    plsc.subcore_barrier()

    @pl.loop(0, RPW)
    def _(j):
        pltpu.sync_copy(ones, accum.at[dstv.at[j]], add=True)

    plsc.subcore_barrier()
    pltpu.sync_copy(accum.at[pl.ds(row0, DRAIN)], out_hbm.at[c, pl.ds(row0, DRAIN)])


def _first_body(x_ref, w_ref, d0_ref, d1_ref, ht_ref, dinv_ref):
    dinv = lax.rsqrt(d0_ref[...] + d1_ref[...] + 1.0)
    dinv_ref[...] = dinv
    ht_ref[...] = jnp.dot(x_ref[...], w_ref[...],
                          preferred_element_type=jnp.float32) * dinv


def _tc_first(x, W1, d0, d1):
    return pl.pallas_call(
        _first_body,
        grid=(GRID,),
        in_specs=[
            pl.BlockSpec((NB, D), lambda i: (i, 0)),
            pl.BlockSpec((D, D), lambda i: (0, 0)),
            pl.BlockSpec((NB, 1), lambda i: (i, 0)),
            pl.BlockSpec((NB, 1), lambda i: (i, 0)),
        ],
        out_specs=[
            pl.BlockSpec((NB, D), lambda i: (i, 0)),
            pl.BlockSpec((NB, 1), lambda i: (i, 0)),
        ],
        out_shape=[
            jax.ShapeDtypeStruct((N, D), jnp.float32),
            jax.ShapeDtypeStruct((N, 1), jnp.float32),
        ],
    )(x, W1, d0, d1)


def _mid_body(s_ref, ht_ref, dinv_ref, b_ref, w_ref, out_ref):
    h = dinv_ref[...] * (s_ref[0] + s_ref[1] + ht_ref[...]) + b_ref[...]
    h = jnp.maximum(h, 0.0)
    out_ref[...] = jnp.dot(h, w_ref[...],
                           preferred_element_type=jnp.float32) * dinv_ref[...]


def _tc_mid(s, ht, dinv, b, W):
    return pl.pallas_call(
        _mid_body,
        grid=(GRID,),
        in_specs=[
            pl.BlockSpec((NC, NB, D), lambda i: (0, i, 0)),
            pl.BlockSpec((NB, D), lambda i: (i, 0)),
            pl.BlockSpec((NB, 1), lambda i: (i, 0)),
            pl.BlockSpec((1, D), lambda i: (0, 0)),
            pl.BlockSpec((D, D), lambda i: (0, 0)),
        ],
        out_specs=pl.BlockSpec((NB, D), lambda i: (i, 0)),
        out_shape=jax.ShapeDtypeStruct((N, D), jnp.float32),
    )(s, ht, dinv, b, W)


def _final_body(s_ref, ht_ref, dinv_ref, b_ref, batch_ref, wl_ref, bl_ref,
                out_ref, sums_ref, cnts_ref):
    i = pl.program_id(0)

    @pl.when(i == 0)
    def _():
        sums_ref[...] = jnp.zeros_like(sums_ref)
        cnts_ref[...] = jnp.zeros_like(cnts_ref)

    h = dinv_ref[...] * (s_ref[0] + s_ref[1] + ht_ref[...]) + b_ref[...]
    oh = (batch_ref[...] == lax.broadcasted_iota(jnp.int32, (NB, G), 1)
          ).astype(jnp.float32)
    sums_ref[...] += lax.dot_general(oh, h, (((0,), (0,)), ((), ())),
                                     preferred_element_type=jnp.float32)
    cnts_ref[...] += lax.dot_general(oh, jnp.ones((NB, 1), jnp.float32),
                                     (((0,), (0,)), ((), ())),
                                     preferred_element_type=jnp.float32)

    @pl.when(i == GRID - 1)
    def _():
        pooled = sums_ref[...] / jnp.maximum(cnts_ref[...], 1.0)
        out_ref[...] = jnp.dot(pooled, wl_ref[...],
                               preferred_element_type=jnp.float32) + bl_ref[...]


def _tc_final(s, ht, dinv, b, batch2d, Wl, bl):
    return pl.pallas_call(
        _final_body,
        grid=(GRID,),
        in_specs=[
            pl.BlockSpec((NC, NB, D), lambda i: (0, i, 0)),
            pl.BlockSpec((NB, D), lambda i: (i, 0)),
            pl.BlockSpec((NB, 1), lambda i: (i, 0)),
            pl.BlockSpec((1, D), lambda i: (0, 0)),
            pl.BlockSpec((NB, 1), lambda i: (i, 0)),
            pl.BlockSpec((D, D_OUT), lambda i: (0, 0)),
            pl.BlockSpec((1, D_OUT), lambda i: (0, 0)),
        ],
        out_specs=pl.BlockSpec((G, D_OUT), lambda i: (0, 0)),
        out_shape=jax.ShapeDtypeStruct((G, D_OUT), jnp.float32),
        scratch_shapes=[
            pltpu.VMEM((G, D), jnp.float32),
            pltpu.VMEM((G, 1), jnp.float32),
        ],
    )(s, ht, dinv, b, batch2d, Wl, bl)


def kernel(x, edge_index, batch, W1, b1, W2, b2, W3, b3, Wl, bl):
    src = edge_index[0]
    dst = edge_index[1]
    pad = E_PAD - E
    srcp = jnp.concatenate([src, jnp.zeros((pad,), jnp.int32)]).reshape(NW * RPW, CHUNK)
    dstp = jnp.concatenate([dst, jnp.full((pad,), N, jnp.int32)]).reshape(NW * RPW, CHUNK)
    zeros_f = jnp.zeros((N_ACC, D), jnp.float32)
    zeros_d = jnp.zeros((N_ACC, LANES), jnp.float32)

    degp = _sc_degree(dstp, zeros_d)
    d0 = degp[0, :N, 0:1]
    d1 = degp[1, :N, 0:1]

    ht1, dinv = _tc_first(x, W1, d0, d1)
    s1 = _sc_scatter(ht1, srcp, dstp, zeros_f)
    ht2 = _tc_mid(s1, ht1, dinv, b1.reshape(1, D), W2)
    s2 = _sc_scatter(ht2, srcp, dstp, zeros_f)
    ht3 = _tc_mid(s2, ht2, dinv, b2.reshape(1, D), W3)
    s3 = _sc_scatter(ht3, srcp, dstp, zeros_f)
    return _tc_final(s3, ht3, dinv, b3.reshape(1, D), batch.reshape(N, 1),
                     Wl, bl.reshape(1, D_OUT))


# speedup vs baseline: 6.9618x; 6.9618x over previous
import functools

import jax
import jax.numpy as jnp
from jax import lax
from jax.experimental import pallas as pl
from jax.experimental.pallas import tpu as pltpu
from jax.experimental.pallas import tpu_sc as plsc

N = 10000
E = 320000
D = 128
D_OUT = 10
G = 64

NC = 2
NS = 16
LANES = 16
NW = NC * NS
CHUNK = 128
RPW = 80
E_PAD = NW * RPW * CHUNK
N_ACC = 10112
DRAIN = N_ACC // NS

NB = 1000
GRID = N // NB


def _sc_scatter_body(ht_hbm, srcp_hbm, dstp_hbm, zeros_hbm, out_hbm,
                     srcv, dstv, buf, accum, sem):
    c = lax.axis_index("c")
    s = lax.axis_index("s")
    wid = s * NC + c
    base = wid * RPW
    pltpu.sync_copy(srcp_hbm.at[pl.ds(base, RPW)], srcv)
    pltpu.sync_copy(dstp_hbm.at[pl.ds(base, RPW)], dstv)
    row0 = s * DRAIN
    pltpu.sync_copy(zeros_hbm.at[pl.ds(row0, DRAIN)], accum.at[pl.ds(row0, DRAIN)])
    plsc.subcore_barrier()

    @pl.loop(0, RPW)
    def _(j):
        pltpu.async_copy(ht_hbm.at[srcv.at[j]], buf, sem).wait()
        pltpu.sync_copy(buf, accum.at[dstv.at[j]], add=True)

    plsc.subcore_barrier()
    pltpu.sync_copy(accum.at[pl.ds(row0, DRAIN)], out_hbm.at[c, pl.ds(row0, DRAIN)])


def _sc_degree_body(dstp_hbm, zeros_hbm, ones_hbm, out_hbm, dstv, ones, accum):
    c = lax.axis_index("c")
    s = lax.axis_index("s")
    wid = s * NC + c
    pltpu.sync_copy(dstp_hbm.at[pl.ds(wid * RPW, RPW)], dstv)
    pltpu.sync_copy(ones_hbm, ones)
    row0 = s * DRAIN
    pltpu.sync_copy(zeros_hbm.at[pl.ds(row0, DRAIN)], accum.at[pl.ds(row0, DRAIN)])
    plsc.subcore_barrier()

    @pl.loop(0, RPW)
    def _(j):
        pltpu.sync_copy(ones, accum.at[dstv.at[j]], add=True)

    plsc.subcore_barrier()
    pltpu.sync_copy(accum.at[pl.ds(row0, DRAIN)], out_hbm.at[c, pl.ds(row0, DRAIN)])


@functools.cache
def _sc_kernels():
    mesh = plsc.VectorSubcoreMesh(core_axis_name="c", subcore_axis_name="s")
    scatter = pl.kernel(
        _sc_scatter_body,
        out_type=jax.ShapeDtypeStruct((NC, N_ACC, D), jnp.float32),
        mesh=mesh,
        scratch_types=[
            pltpu.VMEM((RPW, CHUNK), jnp.int32),
            pltpu.VMEM((RPW, CHUNK), jnp.int32),
            pltpu.VMEM((CHUNK, D), jnp.float32),
            pltpu.VMEM_SHARED((N_ACC, D), jnp.float32),
            pltpu.SemaphoreType.DMA,
        ],
    )
    degree = pl.kernel(
        _sc_degree_body,
        out_type=jax.ShapeDtypeStruct((NC, N_ACC, D), jnp.float32),
        mesh=mesh,
        scratch_types=[
            pltpu.VMEM((RPW, CHUNK), jnp.int32),
            pltpu.VMEM((CHUNK, D), jnp.float32),
            pltpu.VMEM_SHARED((N_ACC, D), jnp.float32),
        ],
    )
    return degree, scatter


def _sc_degree(dstp, zeros_d):
    return _sc_kernels()[0](dstp, zeros_d, jnp.ones((CHUNK, D), jnp.float32))


def _sc_scatter(ht, srcp, dstp, zeros_f):
    return _sc_kernels()[1](ht, srcp, dstp, zeros_f)


def _first_body(x_ref, w_ref, d0_ref, d1_ref, ht_ref, dinv_ref):
    dinv = lax.rsqrt(d0_ref[...] + d1_ref[...] + 1.0)
    dinv_ref[...] = dinv
    ht_ref[...] = jnp.dot(x_ref[...], w_ref[...],
                          preferred_element_type=jnp.float32) * dinv


def _tc_first(x, W1, d0, d1):
    return pl.pallas_call(
        _first_body,
        grid=(GRID,),
        in_specs=[
            pl.BlockSpec((NB, D), lambda i: (i, 0)),
            pl.BlockSpec((D, D), lambda i: (0, 0)),
            pl.BlockSpec((NB, 1), lambda i: (i, 0)),
            pl.BlockSpec((NB, 1), lambda i: (i, 0)),
        ],
        out_specs=[
            pl.BlockSpec((NB, D), lambda i: (i, 0)),
            pl.BlockSpec((NB, 1), lambda i: (i, 0)),
        ],
        out_shape=[
            jax.ShapeDtypeStruct((N, D), jnp.float32),
            jax.ShapeDtypeStruct((N, 1), jnp.float32),
        ],
    )(x, W1, d0, d1)


def _mid_body(s_ref, ht_ref, dinv_ref, b_ref, w_ref, out_ref):
    h = dinv_ref[...] * (s_ref[0] + s_ref[1] + ht_ref[...]) + b_ref[...]
    h = jnp.maximum(h, 0.0)
    out_ref[...] = jnp.dot(h, w_ref[...],
                           preferred_element_type=jnp.float32) * dinv_ref[...]


def _tc_mid(s, ht, dinv, b, W):
    return pl.pallas_call(
        _mid_body,
        grid=(GRID,),
        in_specs=[
            pl.BlockSpec((NC, NB, D), lambda i: (0, i, 0)),
            pl.BlockSpec((NB, D), lambda i: (i, 0)),
            pl.BlockSpec((NB, 1), lambda i: (i, 0)),
            pl.BlockSpec((1, D), lambda i: (0, 0)),
            pl.BlockSpec((D, D), lambda i: (0, 0)),
        ],
        out_specs=pl.BlockSpec((NB, D), lambda i: (i, 0)),
        out_shape=jax.ShapeDtypeStruct((N, D), jnp.float32),
    )(s, ht, dinv, b, W)


def _final_body(s_ref, ht_ref, dinv_ref, b_ref, batch_ref, wl_ref, bl_ref,
                out_ref, sums_ref, cnts_ref):
    i = pl.program_id(0)

    @pl.when(i == 0)
    def _():
        sums_ref[...] = jnp.zeros_like(sums_ref)
        cnts_ref[...] = jnp.zeros_like(cnts_ref)

    h = dinv_ref[...] * (s_ref[0] + s_ref[1] + ht_ref[...]) + b_ref[...]
    oh = (batch_ref[...] == lax.broadcasted_iota(jnp.int32, (NB, G), 1)
          ).astype(jnp.float32)
    sums_ref[...] += lax.dot_general(oh, h, (((0,), (0,)), ((), ())),
                                     preferred_element_type=jnp.float32)
    cnts_ref[...] += lax.dot_general(oh, jnp.ones((NB, 1), jnp.float32),
                                     (((0,), (0,)), ((), ())),
                                     preferred_element_type=jnp.float32)

    @pl.when(i == GRID - 1)
    def _():
        pooled = sums_ref[...] / jnp.maximum(cnts_ref[...], 1.0)
        out_ref[...] = jnp.dot(pooled, wl_ref[...],
                               preferred_element_type=jnp.float32) + bl_ref[...]


def _tc_final(s, ht, dinv, b, batch2d, Wl, bl):
    return pl.pallas_call(
        _final_body,
        grid=(GRID,),
        in_specs=[
            pl.BlockSpec((NC, NB, D), lambda i: (0, i, 0)),
            pl.BlockSpec((NB, D), lambda i: (i, 0)),
            pl.BlockSpec((NB, 1), lambda i: (i, 0)),
            pl.BlockSpec((1, D), lambda i: (0, 0)),
            pl.BlockSpec((NB, 1), lambda i: (i, 0)),
            pl.BlockSpec((D, D_OUT), lambda i: (0, 0)),
            pl.BlockSpec((1, D_OUT), lambda i: (0, 0)),
        ],
        out_specs=pl.BlockSpec((G, D_OUT), lambda i: (0, 0)),
        out_shape=jax.ShapeDtypeStruct((G, D_OUT), jnp.float32),
        scratch_shapes=[
            pltpu.VMEM((G, D), jnp.float32),
            pltpu.VMEM((G, 1), jnp.float32),
        ],
    )(s, ht, dinv, b, batch2d, Wl, bl)


def kernel(x, edge_index, batch, W1, b1, W2, b2, W3, b3, Wl, bl):
    src = edge_index[0]
    dst = edge_index[1]
    pad = E_PAD - E
    srcp = jnp.concatenate([src, jnp.zeros((pad,), jnp.int32)]).reshape(NW * RPW, CHUNK)
    dstp = jnp.concatenate([dst, jnp.full((pad,), N, jnp.int32)]).reshape(NW * RPW, CHUNK)
    zeros_f = jnp.zeros((N_ACC, D), jnp.float32)
    zeros_d = jnp.zeros((N_ACC, D), jnp.float32)

    degp = _sc_degree(dstp, zeros_d)
    d0 = degp[0, :N, 0:1]
    d1 = degp[1, :N, 0:1]

    ht1, dinv = _tc_first(x, W1, d0, d1)
    s1 = _sc_scatter(ht1, srcp, dstp, zeros_f)
    ht2 = _tc_mid(s1, ht1, dinv, b1.reshape(1, D), W2)
    s2 = _sc_scatter(ht2, srcp, dstp, zeros_f)
    ht3 = _tc_mid(s2, ht2, dinv, b2.reshape(1, D), W3)
    s3 = _sc_scatter(ht3, srcp, dstp, zeros_f)
    return _tc_final(s3, ht3, dinv, b3.reshape(1, D), batch.reshape(N, 1),
                     Wl, bl.reshape(1, D_OUT))


# speedup vs baseline: 7.1118x; 1.0215x over previous
import functools

import jax
import jax.numpy as jnp
from jax import lax
from jax.experimental import pallas as pl
from jax.experimental.pallas import tpu as pltpu
from jax.experimental.pallas import tpu_sc as plsc

N = 10000
E = 320000
D = 128
D_OUT = 10
G = 64

NC = 2
NS = 16
LANES = 16
NW = NC * NS
CHUNK = 128
RPW = 80
E_PAD = NW * RPW * CHUNK
N_ACC = 10008
DRAIN = 624
TAIL = N_ACC - NS * DRAIN

NB = 1000
GRID = N // NB


NBUF = 3
IDEPTH = 4
UNROLL = 12


def _sc_scatter_body(ht_hbm, src1_hbm, dst1_hbm, zeros_hbm, out_hbm,
                     isrc, idst, dbuf, accum, issem, idsem, gsem, ssem):
    c = lax.axis_index("c")
    s = lax.axis_index("s")
    wid = s * NC + c
    ebase = wid * RPW * CHUNK
    row0 = s * DRAIN
    pltpu.sync_copy(zeros_hbm.at[pl.ds(row0, DRAIN)], accum.at[pl.ds(row0, DRAIN)])

    @pl.when(s == NS - 1)
    def _():
        pltpu.sync_copy(zeros_hbm.at[pl.ds(NS * DRAIN, TAIL)],
                        accum.at[pl.ds(NS * DRAIN, TAIL)])

    def idx_load(t, m):
        pltpu.async_copy(src1_hbm.at[pl.ds(ebase + t * CHUNK, CHUNK)],
                         isrc.at[m], issem.at[m])
        pltpu.async_copy(dst1_hbm.at[pl.ds(ebase + t * CHUNK, CHUNK)],
                         idst.at[m], idsem.at[m])

    def idx_wait(t, m):
        pltpu.make_async_copy(src1_hbm.at[pl.ds(ebase + t * CHUNK, CHUNK)],
                              isrc.at[m], issem.at[m]).wait()
        pltpu.make_async_copy(dst1_hbm.at[pl.ds(ebase + t * CHUNK, CHUNK)],
                              idst.at[m], idsem.at[m]).wait()

    def gather(m, k):
        pltpu.async_copy(ht_hbm.at[isrc.at[m]], dbuf.at[k], gsem.at[k])

    def gather_wait(m, k):
        pltpu.make_async_copy(ht_hbm.at[isrc.at[m]], dbuf.at[k],
                              gsem.at[k]).wait()

    def scatter(m, k):
        pltpu.async_copy(dbuf.at[k], accum.at[idst.at[m]], ssem.at[k],
                         add=True)

    def scatter_wait(m, k):
        pltpu.make_async_copy(dbuf.at[k], accum.at[idst.at[m]],
                              ssem.at[k]).wait()

    plsc.subcore_barrier()

    def pipe_step(t, b):
        k = b % NBUF
        mi = b % IDEPTH
        gather_wait(mi, k)
        scatter(mi, k)
        if isinstance(t, int):
            if t >= 1:
                scatter_wait((b - 1) % IDEPTH, (b - 1) % NBUF)
            if t + 3 < RPW:
                idx_load(t + 3, (b + 3) % IDEPTH)
            if t + 2 < RPW:
                idx_wait(t + 2, (b + 2) % IDEPTH)
                gather((b + 2) % IDEPTH, (b + 2) % NBUF)
        else:
            scatter_wait((b - 1) % IDEPTH, (b - 1) % NBUF)
            idx_load(t + 3, (b + 3) % IDEPTH)
            idx_wait(t + 2, (b + 2) % IDEPTH)
            gather((b + 2) % IDEPTH, (b + 2) % NBUF)

    for t in range(NBUF):
        idx_load(t, t)
    idx_wait(0, 0)
    gather(0, 0)
    idx_wait(1, 1)
    gather(1, 1)

    for t in range(UNROLL):
        pipe_step(t, t)

    @pl.loop(UNROLL, 72, step=UNROLL)
    def _(j):
        for b in range(UNROLL):
            pipe_step(j + b, b)

    for t in range(72, RPW):
        pipe_step(t, t % UNROLL)
    scatter_wait((RPW - 1) % IDEPTH, (RPW - 1) % NBUF)

    plsc.subcore_barrier()
    pltpu.sync_copy(accum.at[pl.ds(row0, DRAIN)], out_hbm.at[c, pl.ds(row0, DRAIN)])

    @pl.when(s == NS - 1)
    def _():
        pltpu.sync_copy(accum.at[pl.ds(NS * DRAIN, TAIL)],
                        out_hbm.at[c, pl.ds(NS * DRAIN, TAIL)])


def _sc_degree_body(dstp_hbm, zeros_hbm, ones_hbm, out_hbm, dstv, ones, accum,
                    dsem):
    c = lax.axis_index("c")
    s = lax.axis_index("s")
    wid = s * NC + c
    pltpu.sync_copy(dstp_hbm.at[pl.ds(wid * RPW, RPW)], dstv)
    pltpu.sync_copy(ones_hbm, ones)
    row0 = s * DRAIN
    pltpu.sync_copy(zeros_hbm.at[pl.ds(row0, DRAIN)], accum.at[pl.ds(row0, DRAIN)])

    @pl.when(s == NS - 1)
    def _():
        pltpu.sync_copy(zeros_hbm.at[pl.ds(NS * DRAIN, TAIL)],
                        accum.at[pl.ds(NS * DRAIN, TAIL)])

    plsc.subcore_barrier()

    @pl.loop(0, RPW, step=8)
    def _(j):
        for b in range(8):
            pltpu.async_copy(ones, accum.at[dstv.at[j + b]], dsem, add=True)
        for b in range(8):
            pltpu.make_async_copy(ones, accum.at[dstv.at[j]], dsem).wait()

    plsc.subcore_barrier()
    pltpu.sync_copy(accum.at[pl.ds(row0, DRAIN)], out_hbm.at[c, pl.ds(row0, DRAIN)])

    @pl.when(s == NS - 1)
    def _():
        pltpu.sync_copy(accum.at[pl.ds(NS * DRAIN, TAIL)],
                        out_hbm.at[c, pl.ds(NS * DRAIN, TAIL)])


@functools.cache
def _sc_kernels():
    mesh = plsc.VectorSubcoreMesh(core_axis_name="c", subcore_axis_name="s")
    scatter = pl.kernel(
        _sc_scatter_body,
        out_type=jax.ShapeDtypeStruct((NC, N_ACC, D), jnp.float32),
        mesh=mesh,
        scratch_types=[
            pltpu.VMEM((IDEPTH, CHUNK), jnp.int32),
            pltpu.VMEM((IDEPTH, CHUNK), jnp.int32),
            pltpu.VMEM((NBUF, CHUNK, D), jnp.float32),
            pltpu.VMEM_SHARED((N_ACC, D), jnp.float32),
            pltpu.SemaphoreType.DMA((IDEPTH,)),
            pltpu.SemaphoreType.DMA((IDEPTH,)),
            pltpu.SemaphoreType.DMA((NBUF,)),
            pltpu.SemaphoreType.DMA((NBUF,)),
        ],
    )
    degree = pl.kernel(
        _sc_degree_body,
        out_type=jax.ShapeDtypeStruct((NC, N_ACC, D), jnp.float32),
        mesh=mesh,
        scratch_types=[
            pltpu.VMEM((RPW, CHUNK), jnp.int32),
            pltpu.VMEM((CHUNK, D), jnp.float32),
            pltpu.VMEM_SHARED((N_ACC, D), jnp.float32),
            pltpu.SemaphoreType.DMA,
        ],
    )
    return degree, scatter


def _sc_degree(dstp, zeros_d):
    return _sc_kernels()[0](dstp, zeros_d, jnp.ones((CHUNK, D), jnp.float32))


def _sc_scatter(ht, src1, dst1, zeros_f):
    return _sc_kernels()[1](ht, src1, dst1, zeros_f)


def _first_body(x_ref, w_ref, d0_ref, d1_ref, ht_ref, dinv_ref):
    dinv = lax.rsqrt(d0_ref[...] + d1_ref[...] + 1.0)
    dinv_ref[...] = dinv
    ht_ref[...] = jnp.dot(x_ref[...], w_ref[...],
                          preferred_element_type=jnp.float32) * dinv


def _tc_first(x, W1, d0, d1):
    return pl.pallas_call(
        _first_body,
        grid=(GRID,),
        in_specs=[
            pl.BlockSpec((NB, D), lambda i: (i, 0)),
            pl.BlockSpec((D, D), lambda i: (0, 0)),
            pl.BlockSpec((NB, 1), lambda i: (i, 0)),
            pl.BlockSpec((NB, 1), lambda i: (i, 0)),
        ],
        out_specs=[
            pl.BlockSpec((NB, D), lambda i: (i, 0)),
            pl.BlockSpec((NB, 1), lambda i: (i, 0)),
        ],
        out_shape=[
            jax.ShapeDtypeStruct((N, D), jnp.float32),
            jax.ShapeDtypeStruct((N, 1), jnp.float32),
        ],
    )(x, W1, d0, d1)


def _mid_body(s_ref, ht_ref, dinv_ref, b_ref, w_ref, out_ref):
    h = dinv_ref[...] * (s_ref[0] + s_ref[1] + ht_ref[...]) + b_ref[...]
    h = jnp.maximum(h, 0.0)
    out_ref[...] = jnp.dot(h, w_ref[...],
                           preferred_element_type=jnp.float32) * dinv_ref[...]


def _tc_mid(s, ht, dinv, b, W):
    return pl.pallas_call(
        _mid_body,
        grid=(GRID,),
        in_specs=[
            pl.BlockSpec((NC, NB, D), lambda i: (0, i, 0)),
            pl.BlockSpec((NB, D), lambda i: (i, 0)),
            pl.BlockSpec((NB, 1), lambda i: (i, 0)),
            pl.BlockSpec((1, D), lambda i: (0, 0)),
            pl.BlockSpec((D, D), lambda i: (0, 0)),
        ],
        out_specs=pl.BlockSpec((NB, D), lambda i: (i, 0)),
        out_shape=jax.ShapeDtypeStruct((N, D), jnp.float32),
    )(s, ht, dinv, b, W)


def _final_body(s_ref, ht_ref, dinv_ref, b_ref, batch_ref, wl_ref, bl_ref,
                out_ref, sums_ref, cnts_ref):
    i = pl.program_id(0)

    @pl.when(i == 0)
    def _():
        sums_ref[...] = jnp.zeros_like(sums_ref)
        cnts_ref[...] = jnp.zeros_like(cnts_ref)

    h = dinv_ref[...] * (s_ref[0] + s_ref[1] + ht_ref[...]) + b_ref[...]
    oh = (batch_ref[...] == lax.broadcasted_iota(jnp.int32, (NB, G), 1)
          ).astype(jnp.float32)
    sums_ref[...] += lax.dot_general(oh, h, (((0,), (0,)), ((), ())),
                                     preferred_element_type=jnp.float32)
    cnts_ref[...] += lax.dot_general(oh, jnp.ones((NB, 1), jnp.float32),
                                     (((0,), (0,)), ((), ())),
                                     preferred_element_type=jnp.float32)

    @pl.when(i == GRID - 1)
    def _():
        pooled = sums_ref[...] / jnp.maximum(cnts_ref[...], 1.0)
        out_ref[...] = jnp.dot(pooled, wl_ref[...],
                               preferred_element_type=jnp.float32) + bl_ref[...]


def _tc_final(s, ht, dinv, b, batch2d, Wl, bl):
    return pl.pallas_call(
        _final_body,
        grid=(GRID,),
        in_specs=[
            pl.BlockSpec((NC, NB, D), lambda i: (0, i, 0)),
            pl.BlockSpec((NB, D), lambda i: (i, 0)),
            pl.BlockSpec((NB, 1), lambda i: (i, 0)),
            pl.BlockSpec((1, D), lambda i: (0, 0)),
            pl.BlockSpec((NB, 1), lambda i: (i, 0)),
            pl.BlockSpec((D, D_OUT), lambda i: (0, 0)),
            pl.BlockSpec((1, D_OUT), lambda i: (0, 0)),
        ],
        out_specs=pl.BlockSpec((G, D_OUT), lambda i: (0, 0)),
        out_shape=jax.ShapeDtypeStruct((G, D_OUT), jnp.float32),
        scratch_shapes=[
            pltpu.VMEM((G, D), jnp.float32),
            pltpu.VMEM((G, 1), jnp.float32),
        ],
    )(s, ht, dinv, b, batch2d, Wl, bl)


def kernel(x, edge_index, batch, W1, b1, W2, b2, W3, b3, Wl, bl):
    src = edge_index[0]
    dst = edge_index[1]
    pad = E_PAD - E
    src1 = jnp.concatenate([src, jnp.zeros((pad,), jnp.int32)])
    dst1 = jnp.concatenate([dst, jnp.full((pad,), N, jnp.int32)])
    dstp2d = dst1.reshape(NW * RPW, CHUNK)
    zeros_f = jnp.zeros((N_ACC, D), jnp.float32)

    degp = _sc_degree(dstp2d, zeros_f)
    d0 = degp[0, :N, 0:1]
    d1 = degp[1, :N, 0:1]

    ht1, dinv = _tc_first(x, W1, d0, d1)
    s1 = _sc_scatter(ht1, src1, dst1, zeros_f)
    ht2 = _tc_mid(s1, ht1, dinv, b1.reshape(1, D), W2)
    s2 = _sc_scatter(ht2, src1, dst1, zeros_f)
    ht3 = _tc_mid(s2, ht2, dinv, b2.reshape(1, D), W3)
    s3 = _sc_scatter(ht3, src1, dst1, zeros_f)
    return _tc_final(s3, ht3, dinv, b3.reshape(1, D), batch.reshape(N, 1),
                     Wl, bl.reshape(1, D_OUT))


# speedup vs baseline: 26.1429x; 3.6760x over previous
import functools

import jax
import jax.numpy as jnp
from jax import lax
from jax.experimental import pallas as pl
from jax.experimental.pallas import tpu as pltpu
from jax.experimental.pallas import tpu_sc as plsc

N = 10000
E = 320000
D = 128
D_OUT = 10
G = 64

NC = 2
NS = 16
LANES = 16
NW = NC * NS
CHUNK = 128
RPW = 80
E_PAD = NW * RPW * CHUNK
N_ACC = 10008
DRAIN = 624
TAIL = N_ACC - NS * DRAIN

NB = 1000
GRID = N // NB


NBUF = 3
IDEPTH = 4
UNROLL = 12


def _sc_scatter_body(ht_hbm, src1_hbm, dst1_hbm, zeros_hbm, out_hbm,
                     isrc, idst, dbuf, accum, issem, idsem, gsem, ssem):
    c = lax.axis_index("c")
    s = lax.axis_index("s")
    wid = s * NC + c
    ebase = wid * RPW * CHUNK
    row0 = s * DRAIN
    pltpu.sync_copy(zeros_hbm.at[pl.ds(row0, DRAIN)], accum.at[pl.ds(row0, DRAIN)])

    @pl.when(s == NS - 1)
    def _():
        pltpu.sync_copy(zeros_hbm.at[pl.ds(NS * DRAIN, TAIL)],
                        accum.at[pl.ds(NS * DRAIN, TAIL)])

    def idx_load(t, m):
        pltpu.async_copy(src1_hbm.at[pl.ds(ebase + t * CHUNK, CHUNK)],
                         isrc.at[m], issem.at[m])
        pltpu.async_copy(dst1_hbm.at[pl.ds(ebase + t * CHUNK, CHUNK)],
                         idst.at[m], idsem.at[m])

    def idx_wait(t, m):
        pltpu.make_async_copy(src1_hbm.at[pl.ds(ebase + t * CHUNK, CHUNK)],
                              isrc.at[m], issem.at[m]).wait()
        pltpu.make_async_copy(dst1_hbm.at[pl.ds(ebase + t * CHUNK, CHUNK)],
                              idst.at[m], idsem.at[m]).wait()

    def gather(m, k):
        pltpu.async_copy(ht_hbm.at[isrc.at[m]], dbuf.at[k], gsem.at[k])

    def gather_wait(m, k):
        pltpu.make_async_copy(ht_hbm.at[isrc.at[m]], dbuf.at[k],
                              gsem.at[k]).wait()

    def scatter(m, k):
        pltpu.async_copy(dbuf.at[k], accum.at[idst.at[m]], ssem.at[k],
                         add=True)

    def scatter_wait(m, k):
        pltpu.make_async_copy(dbuf.at[k], accum.at[idst.at[m]],
                              ssem.at[k]).wait()

    plsc.subcore_barrier()

    def pipe_step(t, b):
        k = b % NBUF
        mi = b % IDEPTH
        gather_wait(mi, k)
        scatter(mi, k)
        if isinstance(t, int):
            if t >= 1:
                scatter_wait((b - 1) % IDEPTH, (b - 1) % NBUF)
            if t + 3 < RPW:
                idx_load(t + 3, (b + 3) % IDEPTH)
            if t + 2 < RPW:
                idx_wait(t + 2, (b + 2) % IDEPTH)
                gather((b + 2) % IDEPTH, (b + 2) % NBUF)
        else:
            scatter_wait((b - 1) % IDEPTH, (b - 1) % NBUF)
            idx_load(t + 3, (b + 3) % IDEPTH)
            idx_wait(t + 2, (b + 2) % IDEPTH)
            gather((b + 2) % IDEPTH, (b + 2) % NBUF)

    for t in range(NBUF):
        idx_load(t, t)
    idx_wait(0, 0)
    gather(0, 0)
    idx_wait(1, 1)
    gather(1, 1)

    for t in range(UNROLL):
        pipe_step(t, t)

    @pl.loop(UNROLL, 72, step=UNROLL)
    def _(j):
        for b in range(UNROLL):
            pipe_step(j + b, b)

    for t in range(72, RPW):
        pipe_step(t, t % UNROLL)
    scatter_wait((RPW - 1) % IDEPTH, (RPW - 1) % NBUF)

    plsc.subcore_barrier()
    pltpu.sync_copy(accum.at[pl.ds(row0, DRAIN)], out_hbm.at[c, pl.ds(row0, DRAIN)])

    @pl.when(s == NS - 1)
    def _():
        pltpu.sync_copy(accum.at[pl.ds(NS * DRAIN, TAIL)],
                        out_hbm.at[c, pl.ds(NS * DRAIN, TAIL)])


def _sc_degree_body(dstp_hbm, zeros_hbm, ones_hbm, out_hbm, dstv, ones, accum,
                    dsem):
    c = lax.axis_index("c")
    s = lax.axis_index("s")
    wid = s * NC + c
    pltpu.sync_copy(dstp_hbm.at[pl.ds(wid * RPW, RPW)], dstv)
    pltpu.sync_copy(ones_hbm, ones)
    row0 = s * DRAIN
    pltpu.sync_copy(zeros_hbm.at[pl.ds(row0, DRAIN)], accum.at[pl.ds(row0, DRAIN)])

    @pl.when(s == NS - 1)
    def _():
        pltpu.sync_copy(zeros_hbm.at[pl.ds(NS * DRAIN, TAIL)],
                        accum.at[pl.ds(NS * DRAIN, TAIL)])

    plsc.subcore_barrier()

    @pl.loop(0, RPW, step=8)
    def _(j):
        for b in range(8):
            pltpu.async_copy(ones, accum.at[dstv.at[j + b]], dsem, add=True)
        for b in range(8):
            pltpu.make_async_copy(ones, accum.at[dstv.at[j]], dsem).wait()

    plsc.subcore_barrier()
    pltpu.sync_copy(accum.at[pl.ds(row0, DRAIN)], out_hbm.at[c, pl.ds(row0, DRAIN)])

    @pl.when(s == NS - 1)
    def _():
        pltpu.sync_copy(accum.at[pl.ds(NS * DRAIN, TAIL)],
                        out_hbm.at[c, pl.ds(NS * DRAIN, TAIL)])


@functools.cache
def _sc_kernels():
    mesh = plsc.VectorSubcoreMesh(core_axis_name="c", subcore_axis_name="s")
    scatter = pl.kernel(
        _sc_scatter_body,
        out_type=jax.ShapeDtypeStruct((NC, N_ACC, D), jnp.float32),
        mesh=mesh,
        scratch_types=[
            pltpu.VMEM((IDEPTH, CHUNK), jnp.int32),
            pltpu.VMEM((IDEPTH, CHUNK), jnp.int32),
            pltpu.VMEM((NBUF, CHUNK, D), jnp.float32),
            pltpu.VMEM_SHARED((N_ACC, D), jnp.float32),
            pltpu.SemaphoreType.DMA((IDEPTH,)),
            pltpu.SemaphoreType.DMA((IDEPTH,)),
            pltpu.SemaphoreType.DMA((NBUF,)),
            pltpu.SemaphoreType.DMA((NBUF,)),
        ],
    )
    degree = pl.kernel(
        _sc_degree_body,
        out_type=jax.ShapeDtypeStruct((NC, N_ACC, D), jnp.float32),
        mesh=mesh,
        scratch_types=[
            pltpu.VMEM((RPW, CHUNK), jnp.int32),
            pltpu.VMEM((CHUNK, D), jnp.float32),
            pltpu.VMEM_SHARED((N_ACC, D), jnp.float32),
            pltpu.SemaphoreType.DMA,
        ],
    )
    return degree, scatter


def _sc_degree(dstp, zeros_d):
    return _sc_kernels()[0](dstp, zeros_d, jnp.ones((CHUNK, D), jnp.float32))


def _sc_scatter(ht, src1, dst1, zeros_f):
    return _sc_kernels()[1](ht, src1, dst1, zeros_f)


def _first_body(x_ref, w_ref, d0_ref, d1_ref, ht_ref, dinv_ref):
    dinv = lax.rsqrt(d0_ref[...] + d1_ref[...] + 1.0)
    dinv_ref[...] = dinv
    ht_ref[...] = jnp.dot(x_ref[...], w_ref[...],
                          preferred_element_type=jnp.float32) * dinv


def _tc_first(x, W1, d0, d1):
    return pl.pallas_call(
        _first_body,
        grid=(GRID,),
        in_specs=[
            pl.BlockSpec((NB, D), lambda i: (i, 0)),
            pl.BlockSpec((D, D), lambda i: (0, 0)),
            pl.BlockSpec((NB, 1), lambda i: (i, 0)),
            pl.BlockSpec((NB, 1), lambda i: (i, 0)),
        ],
        out_specs=[
            pl.BlockSpec((NB, D), lambda i: (i, 0)),
            pl.BlockSpec((NB, 1), lambda i: (i, 0)),
        ],
        out_shape=[
            jax.ShapeDtypeStruct((N, D), jnp.float32),
            jax.ShapeDtypeStruct((N, 1), jnp.float32),
        ],
    )(x, W1, d0, d1)


def _mid_body(s_ref, ht_ref, dinv_ref, b_ref, w_ref, out_ref):
    h = dinv_ref[...] * (s_ref[0] + s_ref[1] + ht_ref[...]) + b_ref[...]
    h = jnp.maximum(h, 0.0)
    out_ref[...] = jnp.dot(h, w_ref[...],
                           preferred_element_type=jnp.float32) * dinv_ref[...]


def _tc_mid(s, ht, dinv, b, W):
    return pl.pallas_call(
        _mid_body,
        grid=(GRID,),
        in_specs=[
            pl.BlockSpec((NC, NB, D), lambda i: (0, i, 0)),
            pl.BlockSpec((NB, D), lambda i: (i, 0)),
            pl.BlockSpec((NB, 1), lambda i: (i, 0)),
            pl.BlockSpec((1, D), lambda i: (0, 0)),
            pl.BlockSpec((D, D), lambda i: (0, 0)),
        ],
        out_specs=pl.BlockSpec((NB, D), lambda i: (i, 0)),
        out_shape=jax.ShapeDtypeStruct((N, D), jnp.float32),
    )(s, ht, dinv, b, W)


def _final_body(s_ref, ht_ref, dinv_ref, b_ref, batch_ref, wl_ref, bl_ref,
                out_ref, sums_ref, cnts_ref):
    i = pl.program_id(0)

    @pl.when(i == 0)
    def _():
        sums_ref[...] = jnp.zeros_like(sums_ref)
        cnts_ref[...] = jnp.zeros_like(cnts_ref)

    h = dinv_ref[...] * (s_ref[0] + s_ref[1] + ht_ref[...]) + b_ref[...]
    oh = (batch_ref[...] == lax.broadcasted_iota(jnp.int32, (NB, G), 1)
          ).astype(jnp.float32)
    sums_ref[...] += lax.dot_general(oh, h, (((0,), (0,)), ((), ())),
                                     preferred_element_type=jnp.float32)
    cnts_ref[...] += lax.dot_general(oh, jnp.ones((NB, 1), jnp.float32),
                                     (((0,), (0,)), ((), ())),
                                     preferred_element_type=jnp.float32)

    @pl.when(i == GRID - 1)
    def _():
        pooled = sums_ref[...] / jnp.maximum(cnts_ref[...], 1.0)
        out_ref[...] = jnp.dot(pooled, wl_ref[...],
                               preferred_element_type=jnp.float32) + bl_ref[...]


def _tc_final(s, ht, dinv, b, batch2d, Wl, bl):
    return pl.pallas_call(
        _final_body,
        grid=(GRID,),
        in_specs=[
            pl.BlockSpec((NC, NB, D), lambda i: (0, i, 0)),
            pl.BlockSpec((NB, D), lambda i: (i, 0)),
            pl.BlockSpec((NB, 1), lambda i: (i, 0)),
            pl.BlockSpec((1, D), lambda i: (0, 0)),
            pl.BlockSpec((NB, 1), lambda i: (i, 0)),
            pl.BlockSpec((D, D_OUT), lambda i: (0, 0)),
            pl.BlockSpec((1, D_OUT), lambda i: (0, 0)),
        ],
        out_specs=pl.BlockSpec((G, D_OUT), lambda i: (0, 0)),
        out_shape=jax.ShapeDtypeStruct((G, D_OUT), jnp.float32),
        scratch_shapes=[
            pltpu.VMEM((G, D), jnp.float32),
            pltpu.VMEM((G, 1), jnp.float32),
        ],
    )(s, ht, dinv, b, batch2d, Wl, bl)


def kernel(x, edge_index, batch, W1, b1, W2, b2, W3, b3, Wl, bl):
    src = edge_index[0]
    dst = edge_index[1]
    pad = E_PAD - E
    pad_iota = jnp.arange(pad, dtype=jnp.int32)
    src1 = jnp.concatenate([src, pad_iota % N])
    dst1 = jnp.concatenate([dst, N + (pad_iota % (N_ACC - N))])
    dstp2d = dst1.reshape(NW * RPW, CHUNK)
    zeros_f = jnp.zeros((N_ACC, D), jnp.float32)

    degp = _sc_degree(dstp2d, zeros_f)
    d0 = degp[0, :N, 0:1]
    d1 = degp[1, :N, 0:1]

    ht1, dinv = _tc_first(x, W1, d0, d1)
    s1 = _sc_scatter(ht1, src1, dst1, zeros_f)
    ht2 = _tc_mid(s1, ht1, dinv, b1.reshape(1, D), W2)
    s2 = _sc_scatter(ht2, src1, dst1, zeros_f)
    ht3 = _tc_mid(s2, ht2, dinv, b2.reshape(1, D), W3)
    s3 = _sc_scatter(ht3, src1, dst1, zeros_f)
    return _tc_final(s3, ht3, dinv, b3.reshape(1, D), batch.reshape(N, 1),
                     Wl, bl.reshape(1, D_OUT))
